# Initial kernel scaffold; baseline (speedup 1.0000x reference)
import jax
import jax.numpy as jnp
from jax.experimental import pallas as pl


def kernel(x, emb_table, fc_w, fc_b):
    raise NotImplementedError("write your pallas kernel here")


# speedup vs baseline: 1.4606x; 1.4606x over previous
import functools

import jax
import jax.numpy as jnp
from jax import lax
from jax.experimental import pallas as pl
from jax.experimental.pallas import tpu as pltpu
from jax.experimental.pallas import tpu_sc as plsc

BATCH = 16384
SEQ = 50
EMBED = 64
NUM_CLASSES = 100

NC, NS = 2, 16
NW = NC * NS
BPW = BATCH // NW
PAIRS = BPW // 2
IDX_PER_GATHER = 2 * SEQ
IDX_PAD = 104
NBUF = 4
LANES = 16
VPR = EMBED // LANES


def _pool_body(x_hbm, tbl_hbm, out_hbm, idx_v, pooled_v, bufs, sems):
    wid = lax.axis_index("c") * NS + lax.axis_index("s")

    pltpu.sync_copy(x_hbm.at[wid], idx_v)

    def fire(g, b):
        return pltpu.async_copy(tbl_hbm.at[idx_v.at[g]], bufs[b], sems[b])

    def wait(g, b):
        pltpu.make_async_copy(tbl_hbm.at[idx_v.at[g]], bufs[b], sems[b]).wait()

    for b in range(NBUF):
        fire(b, b)

    @pl.loop(0, PAIRS, step=NBUF)
    def _(g0):
        for b in range(NBUF):
            g = g0 + b
            wait(g, b)
            buf = bufs[b]
            for be in range(2):
                accs = [buf[be * SEQ, pl.ds(c * LANES, LANES)] for c in range(VPR)]
                for j in range(1, SEQ):
                    for c in range(VPR):
                        accs[c] += buf[be * SEQ + j, pl.ds(c * LANES, LANES)]
                row = 2 * g + be
                for c in range(VPR):
                    pooled_v[row, pl.ds(c * LANES, LANES)] = accs[c] * (1.0 / SEQ)

            @pl.when(g + NBUF < PAIRS)
            def _():
                fire(g + NBUF, b)

    pltpu.sync_copy(pooled_v, out_hbm.at[pl.ds(wid * BPW, BPW)])


def _sc_pool(xr, emb_table):
    scratch = [
        pltpu.VMEM((PAIRS, IDX_PAD), jnp.int32),
        pltpu.VMEM((BPW, EMBED), jnp.float32),
        [pltpu.VMEM((IDX_PAD, EMBED), jnp.float32) for _ in range(NBUF)],
        [pltpu.SemaphoreType.DMA for _ in range(NBUF)],
    ]
    k = pl.kernel(
        _pool_body,
        out_type=jax.ShapeDtypeStruct((BATCH, EMBED), jnp.float32),
        mesh=plsc.VectorSubcoreMesh(
            core_axis_name="c", subcore_axis_name="s",
            num_cores=NC, num_subcores=NS,
        ),
        scratch_types=scratch,
        compiler_params=pltpu.CompilerParams(use_tc_tiling_on_sc=False),
    )
    return k(xr, emb_table)


def _mm_body(p_ref, w_ref, b_ref, o_ref):
    o_ref[...] = (
        jnp.dot(p_ref[...], w_ref[...], preferred_element_type=jnp.float32)
        + b_ref[...]
    )


def _fc(pooled, fc_w, fc_b2):
    blk = 1024
    grid = (BATCH // blk,)
    return pl.pallas_call(
        _mm_body,
        grid=grid,
        in_specs=[
            pl.BlockSpec((blk, EMBED), lambda i: (i, 0)),
            pl.BlockSpec((EMBED, NUM_CLASSES), lambda i: (0, 0)),
            pl.BlockSpec((1, NUM_CLASSES), lambda i: (0, 0)),
        ],
        out_specs=pl.BlockSpec((blk, NUM_CLASSES), lambda i: (i, 0)),
        out_shape=jax.ShapeDtypeStruct((BATCH, NUM_CLASSES), jnp.float32),
    )(pooled, fc_w, fc_b2)


def kernel(x, emb_table, fc_w, fc_b):
    xr = x.reshape(NW, PAIRS, IDX_PER_GATHER)
    xr = jnp.pad(xr, ((0, 0), (0, 0), (0, IDX_PAD - IDX_PER_GATHER)))
    pooled = _sc_pool(xr, emb_table)
    return _fc(pooled, fc_w, fc_b.reshape(1, NUM_CLASSES))


# speedup vs baseline: 2.3262x; 1.5927x over previous
import jax
import jax.numpy as jnp
from jax import lax
from jax.experimental import pallas as pl
from jax.experimental.pallas import tpu as pltpu
from jax.experimental.pallas import tpu_sc as plsc

BATCH = 16384
SEQ = 50
EMBED = 64
NUM_CLASSES = 100

NC, NS = 2, 16
NW = NC * NS
BPW = BATCH // NW
BPG = 4
IDX_PER_GATHER = BPG * SEQ
NGATHER = BPW // BPG
NBUF = 4
LANES = 16
VPR = EMBED // LANES


def _pool_body(x_hbm, tbl_hbm, out_hbm, idx_v, pooled_v, bufs, sems):
    wid = lax.axis_index("c") * NS + lax.axis_index("s")

    pltpu.sync_copy(x_hbm.at[wid], idx_v)

    def fire(g, b):
        return pltpu.async_copy(tbl_hbm.at[idx_v.at[g]], bufs[b], sems[b])

    def wait(g, b):
        pltpu.make_async_copy(tbl_hbm.at[idx_v.at[g]], bufs[b], sems[b]).wait()

    for b in range(NBUF):
        fire(b, b)

    @pl.loop(0, NGATHER, step=NBUF)
    def _(g0):
        for b in range(NBUF):
            g = g0 + b
            wait(g, b)
            buf = bufs[b]
            for be in range(BPG):
                accs = [buf[be * SEQ, pl.ds(c * LANES, LANES)] for c in range(VPR)]
                for j in range(1, SEQ):
                    for c in range(VPR):
                        accs[c] += buf[be * SEQ + j, pl.ds(c * LANES, LANES)]
                row = BPG * g + be
                for c in range(VPR):
                    pooled_v[row, pl.ds(c * LANES, LANES)] = accs[c] * (1.0 / SEQ)

            @pl.when(g + NBUF < NGATHER)
            def _():
                fire(g + NBUF, b)

    pltpu.sync_copy(pooled_v, out_hbm.at[pl.ds(wid * BPW, BPW)])


def _sc_pool(xr, emb_table):
    scratch = [
        pltpu.VMEM((NGATHER, IDX_PER_GATHER), jnp.int32),
        pltpu.VMEM((BPW, EMBED), jnp.float32),
        [pltpu.VMEM((IDX_PER_GATHER, EMBED), jnp.float32) for _ in range(NBUF)],
        [pltpu.SemaphoreType.DMA for _ in range(NBUF)],
    ]
    k = pl.kernel(
        _pool_body,
        out_type=jax.ShapeDtypeStruct((BATCH, EMBED), jnp.float32),
        mesh=plsc.VectorSubcoreMesh(
            core_axis_name="c", subcore_axis_name="s",
            num_cores=NC, num_subcores=NS,
        ),
        scratch_types=scratch,
        compiler_params=pltpu.CompilerParams(use_tc_tiling_on_sc=False),
    )
    return k(xr, emb_table)


def _mm_body(p_ref, w_ref, b_ref, o_ref):
    o_ref[...] = (
        jnp.dot(p_ref[...], w_ref[...], preferred_element_type=jnp.float32)
        + b_ref[...]
    )


def _fc(pooled, fc_w, fc_b2):
    blk = 1024
    grid = (BATCH // blk,)
    return pl.pallas_call(
        _mm_body,
        grid=grid,
        in_specs=[
            pl.BlockSpec((blk, EMBED), lambda i: (i, 0)),
            pl.BlockSpec((EMBED, NUM_CLASSES), lambda i: (0, 0)),
            pl.BlockSpec((1, NUM_CLASSES), lambda i: (0, 0)),
        ],
        out_specs=pl.BlockSpec((blk, NUM_CLASSES), lambda i: (i, 0)),
        out_shape=jax.ShapeDtypeStruct((BATCH, NUM_CLASSES), jnp.float32),
    )(pooled, fc_w, fc_b2)


def kernel(x, emb_table, fc_w, fc_b):
    xr = x.reshape(NW, NGATHER, IDX_PER_GATHER)
    pooled = _sc_pool(xr, emb_table)
    return _fc(pooled, fc_w, fc_b.reshape(1, NUM_CLASSES))
